# baseline (speedup 1.0000x reference)
import jax
import jax.numpy as jnp
from jax.experimental import pallas as pl
from jax.experimental.pallas import tpu as pltpu

_N = 65536
_D = 64
_K = 512
_ITERS = 4
_T = 4096
_NT = _N // _T

_F32 = jnp.float32
_BF16 = jnp.bfloat16


def _dot(a, b, dims):
    return jax.lax.dot_general(a, b, (dims, ((), ())),
                               preferred_element_type=_F32)


def _body(x_ref, out_ref, centers_ref, sums_ref, counts_ref, acc_ref):
    centers_ref[...] = x_ref[0:_K, :]
    acc_ref[...] = jnp.zeros((1, 1), _F32)
    iota_f = jax.lax.broadcasted_iota(jnp.int32, (_T, _K), 1).astype(_F32)

    for p in range(_ITERS + 1):
        final = p == _ITERS
        c = centers_ref[...]
        cm2 = -2.0 * c
        cn = jnp.sum(c * c, axis=1, keepdims=True).reshape(1, _K)
        sums_ref[...] = jnp.zeros((_K, _D), _F32)
        counts_ref[...] = jnp.zeros((1, _K), _F32)

        def tile(t, carry):
            xt = x_ref[pl.ds(t * _T, _T), :]
            q = _dot(xt, cm2, ((1,), (1,)))
            xn = jnp.sum(xt * xt, axis=1, keepdims=True)
            d2 = (xn + q) + cn
            rowmin = jnp.min(d2, axis=1, keepdims=True)
            cid = jnp.min(jnp.where(d2 == rowmin, iota_f, _F32(_K)), axis=1,
                          keepdims=True)
            oh = (iota_f == cid).astype(_BF16)
            xh = xt.astype(_BF16)
            r1 = xt - xh.astype(_F32)
            xm = r1.astype(_BF16)
            xl = (r1 - xm.astype(_F32)).astype(_BF16)
            seg = (_dot(oh, xh, ((0,), (0,))) + _dot(oh, xm, ((0,), (0,)))
                   + _dot(oh, xl, ((0,), (0,))))
            sums_ref[...] += seg
            counts_ref[...] += jnp.sum(oh.astype(_F32), axis=0,
                                       keepdims=True)
            if final:
                logits = -0.5 * q
                m = jnp.max(logits, axis=1, keepdims=True)
                lse = m + jnp.log(
                    jnp.sum(jnp.exp(logits - m), axis=1, keepdims=True))
                acc_ref[...] = acc_ref[...] + jnp.sum(lse)
            return carry

        jax.lax.fori_loop(0, _NT, tile, 0)

        if not final:
            cnt = counts_ref[...].reshape(_K, 1)
            newc = sums_ref[...] / jnp.maximum(cnt, 1.0)
            centers_ref[...] = jnp.where(cnt > 0.0, newc, c)

    cnt = counts_ref[...].reshape(_K, 1)
    occ = (cnt > 0.0).astype(_F32)
    ki = jax.lax.broadcasted_iota(jnp.int32, (_K, _K), 0)
    ji = jax.lax.broadcasted_iota(jnp.int32, (_K, _K), 1)
    tril = (ji < ki).astype(_F32)
    rank = _dot(tril, occ, ((1,), (0,)))
    rank_i = rank.astype(jnp.int32)
    oh_rank = (rank_i == ji).astype(_F32)
    c_rank = _dot(oh_rank, centers_ref[...], ((1,), (0,)))
    picked_sum = jnp.sum(sums_ref[...] * c_rank)
    out_ref[...] = (acc_ref[...] - picked_sum) / _N


def kernel(x):
    out = pl.pallas_call(
        _body,
        out_shape=jax.ShapeDtypeStruct((1, 1), _F32),
        scratch_shapes=[
            pltpu.VMEM((_K, _D), _F32),
            pltpu.VMEM((_K, _D), _F32),
            pltpu.VMEM((1, _K), _F32),
            pltpu.VMEM((1, 1), _F32),
        ],
    )(x)
    return out[0, 0]
